# baseline (speedup 1.0000x reference)
import functools

import jax
import jax.numpy as jnp
from jax import lax
from jax.experimental import pallas as pl
from jax.experimental.pallas import tpu as pltpu
from jax.experimental.pallas import tpu_sc as plsc

N_E = 8192
E_DIM = 256
BETA = 0.25

M = 8192
BM = 256
BN = 2048
NJ = N_E // BN


RB = 64


def _esum_body(e_ref, out_ref):
    eb = e_ref[...]
    ones = jnp.ones((1, E_DIM), jnp.float32)
    out_ref[...] = lax.dot_general(ones, eb * eb, (((1,), (1,)), ((), ())),
                                   preferred_element_type=jnp.float32)


def _esum(emb):
    bn = 1024
    return pl.pallas_call(
        _esum_body,
        grid=(N_E // bn,),
        in_specs=[pl.BlockSpec((bn, E_DIM), lambda i: (i, 0))],
        out_specs=pl.BlockSpec((1, bn), lambda i: (0, i)),
        out_shape=jax.ShapeDtypeStruct((1, N_E), jnp.float32),
    )(emb)


def _dist_argmin_body(z_ref, e_ref, es_ref, d_ref, idx_ref):
    i = pl.program_id(0)
    zc = z_ref[0, :, pl.ds((i % 4) * BM, BM)]
    zb = jnp.transpose(zc)
    eb = e_ref[...]

    zb2 = (zb + zb).astype(jnp.bfloat16)
    mm2 = lax.dot_general(zb2, eb, (((1,), (1,)), ((), ())),
                          preferred_element_type=jnp.float32)
    zsum = jnp.sum(zb * zb, axis=1, keepdims=True)
    esum = es_ref[...]

    nc = N_E // 128
    for r in range(BM // RB):
        rl, rh = r * RB, (r + 1) * RB
        zs = zsum[rl:rh]
        acc_v = jnp.full((RB, 128), jnp.inf, jnp.float32)
        acc_i = jnp.zeros((RB, 128), jnp.int32)
        for c in range(nc):
            lo, hi = c * 128, (c + 1) * 128
            v = (zs + esum[:, lo:hi]) - mm2[rl:rh, lo:hi]
            d_ref[rl:rh, lo:hi] = v
            m = v < acc_v
            acc_i = jnp.where(m, jnp.full((RB, 128), c, jnp.int32), acc_i)
            acc_v = jnp.where(m, v, acc_v)
        rowmin = jnp.min(acc_v, axis=1, keepdims=True)
        gidx = acc_i * 128 + lax.broadcasted_iota(jnp.int32, (RB, 128), 1)
        idx_ref[rl:rh] = jnp.min(
            jnp.where(acc_v == rowmin, gidx, jnp.int32(2**30)),
            axis=1, keepdims=True)


def _dist_argmin(z4, emb_bf, esum):
    return pl.pallas_call(
        _dist_argmin_body,
        grid=(M // BM,),
        in_specs=[
            pl.BlockSpec((1, E_DIM, 1024), lambda i: (i // 4, 0, 0)),
            pl.BlockSpec((N_E, E_DIM), lambda i: (0, 0)),
            pl.BlockSpec((1, N_E), lambda i: (0, 0)),
        ],
        out_specs=[
            pl.BlockSpec((BM, N_E), lambda i: (i, 0)),
            pl.BlockSpec((BM, 1), lambda i: (i, 0)),
        ],
        out_shape=[
            jax.ShapeDtypeStruct((M, N_E), jnp.float32),
            jax.ShapeDtypeStruct((M, 1), jnp.int32),
        ],
    )(z4, emb_bf, esum)


def _sc_gather(emb, idx):
    info = plsc.get_sparse_core_info()
    nw = info.num_cores * info.num_subcores
    bw = M // nw
    mesh = plsc.VectorSubcoreMesh(core_axis_name="c", subcore_axis_name="s")

    @functools.partial(
        pl.kernel, mesh=mesh,
        out_type=jax.ShapeDtypeStruct((M, E_DIM), jnp.float32),
        scratch_types=[
            pltpu.VMEM((bw,), jnp.int32),
            pltpu.VMEM((bw, E_DIM), jnp.float32),
            pltpu.SemaphoreType.DMA,
        ],
    )
    def gather_k(table_hbm, idx_hbm, out_hbm, idx_v, rows_v, sem):
        wid = lax.axis_index("s") * info.num_cores + lax.axis_index("c")
        base = wid * bw
        pltpu.sync_copy(idx_hbm.at[pl.ds(base, bw)], idx_v)
        copies = []
        for c in range(bw // 128):
            copies.append(pltpu.async_copy(
                table_hbm.at[idx_v.at[pl.ds(c * 128, 128)]],
                rows_v.at[pl.ds(c * 128, 128)], sem))
        for cp in copies:
            cp.wait()
        pltpu.sync_copy(rows_v, out_hbm.at[pl.ds(base, bw)])

    return gather_k(emb, idx)


def _loss_st_body(zp_ref, zq_ref, o_ref, loss_ref, acc_ref):
    i = pl.program_id(0)
    nb = pl.num_programs(0)
    zp = zp_ref[...].reshape(E_DIM, 1024)
    zq = jnp.transpose(zq_ref[...])
    diff = zq - zp
    o_ref[...] = (zp + diff).reshape(1, E_DIM, 1024)
    s = jnp.sum(diff * diff)

    @pl.when(i == 0)
    def _():
        acc_ref[0] = s

    @pl.when(i > 0)
    def _():
        acc_ref[0] += s

    @pl.when(i == nb - 1)
    def _():
        m = acc_ref[0] / jnp.float32(M * E_DIM)
        loss_ref[0, 0] = m + jnp.float32(BETA) * m


def _loss_st(z4, zq_flat):
    bm = 1024
    return pl.pallas_call(
        _loss_st_body,
        grid=(M // bm,),
        in_specs=[
            pl.BlockSpec((1, E_DIM, bm), lambda i: (i, 0, 0)),
            pl.BlockSpec((bm, E_DIM), lambda i: (i, 0)),
        ],
        out_specs=[
            pl.BlockSpec((1, E_DIM, bm), lambda i: (i, 0, 0)),
            pl.BlockSpec((1, 1), lambda i: (0, 0), memory_space=pltpu.SMEM),
        ],
        out_shape=[
            jax.ShapeDtypeStruct((8, E_DIM, bm), jnp.float32),
            jax.ShapeDtypeStruct((1, 1), jnp.float32),
        ],
        scratch_shapes=[pltpu.SMEM((1,), jnp.float32)],
    )(z4, zq_flat)


def kernel(z, embedding_weight):
    z4 = z.reshape(8, E_DIM, 1024)
    esum = _esum(embedding_weight)
    d, idx2 = _dist_argmin(z4, embedding_weight.astype(jnp.bfloat16), esum)
    idx = idx2.reshape(M)
    zq_flat = _sc_gather(embedding_weight, idx)
    out4, loss2 = _loss_st(z4, zq_flat)
    z_q = out4.reshape(z.shape)
    return (z_q, loss2[0, 0], idx, d)


# speedup vs baseline: 1.0490x; 1.0490x over previous
import functools

import jax
import jax.numpy as jnp
from jax import lax
from jax.experimental import pallas as pl
from jax.experimental.pallas import tpu as pltpu
from jax.experimental.pallas import tpu_sc as plsc

N_E = 8192
E_DIM = 256
BETA = 0.25

M = 8192
BM = 256
BN = 2048
NJ = N_E // BN


RB = 64


def _esum_body(e_ref, out_ref):
    eb = e_ref[...]
    ones = jnp.ones((1, E_DIM), jnp.float32)
    out_ref[...] = lax.dot_general(ones, eb * eb, (((1,), (1,)), ((), ())),
                                   preferred_element_type=jnp.float32)


def _esum(emb):
    bn = 1024
    return pl.pallas_call(
        _esum_body,
        grid=(N_E // bn,),
        in_specs=[pl.BlockSpec((bn, E_DIM), lambda i: (i, 0))],
        out_specs=pl.BlockSpec((1, bn), lambda i: (0, i)),
        out_shape=jax.ShapeDtypeStruct((1, N_E), jnp.float32),
    )(emb)


def _dist_argmin_body(z_ref, e_ref, es_ref, d_ref, idx_ref):
    zb = z_ref[...]
    eb = e_ref[...]

    zb2 = (zb + zb).astype(jnp.bfloat16)
    mm2 = lax.dot_general(zb2, eb, (((1,), (1,)), ((), ())),
                          preferred_element_type=jnp.float32)
    zsum = jnp.sum(zb * zb, axis=1, keepdims=True)
    esum = es_ref[...]

    nc = N_E // 128
    for r in range(BM // RB):
        rl, rh = r * RB, (r + 1) * RB
        zs = zsum[rl:rh]
        acc_v = jnp.full((RB, 128), jnp.inf, jnp.float32)
        acc_i = jnp.zeros((RB, 128), jnp.int32)
        for c in range(nc):
            lo, hi = c * 128, (c + 1) * 128
            v = (zs + esum[:, lo:hi]) - mm2[rl:rh, lo:hi]
            d_ref[rl:rh, lo:hi] = v
            m = v < acc_v
            acc_i = jnp.where(m, jnp.full((RB, 128), c, jnp.int32), acc_i)
            acc_v = jnp.where(m, v, acc_v)
        rowmin = jnp.min(acc_v, axis=1, keepdims=True)
        gidx = acc_i * 128 + lax.broadcasted_iota(jnp.int32, (RB, 128), 1)
        idx_ref[rl:rh] = jnp.min(
            jnp.where(acc_v == rowmin, gidx, jnp.int32(2**30)),
            axis=1, keepdims=True)


def _dist_argmin(z_flat, emb_bf, esum):
    return pl.pallas_call(
        _dist_argmin_body,
        grid=(M // BM,),
        in_specs=[
            pl.BlockSpec((BM, E_DIM), lambda i: (i, 0)),
            pl.BlockSpec((N_E, E_DIM), lambda i: (0, 0)),
            pl.BlockSpec((1, N_E), lambda i: (0, 0)),
        ],
        out_specs=[
            pl.BlockSpec((BM, N_E), lambda i: (i, 0)),
            pl.BlockSpec((BM, 1), lambda i: (i, 0)),
        ],
        out_shape=[
            jax.ShapeDtypeStruct((M, N_E), jnp.float32),
            jax.ShapeDtypeStruct((M, 1), jnp.int32),
        ],
    )(z_flat, emb_bf, esum)


def _sc_gather(emb, idx):
    info = plsc.get_sparse_core_info()
    nw = info.num_cores * info.num_subcores
    bw = M // nw
    mesh = plsc.VectorSubcoreMesh(core_axis_name="c", subcore_axis_name="s")

    @functools.partial(
        pl.kernel, mesh=mesh,
        out_type=jax.ShapeDtypeStruct((M, E_DIM), jnp.float32),
        scratch_types=[
            pltpu.VMEM((bw,), jnp.int32),
            pltpu.VMEM((bw, E_DIM), jnp.float32),
            pltpu.SemaphoreType.DMA,
        ],
    )
    def gather_k(table_hbm, idx_hbm, out_hbm, idx_v, rows_v, sem):
        wid = lax.axis_index("s") * info.num_cores + lax.axis_index("c")
        base = wid * bw
        pltpu.sync_copy(idx_hbm.at[pl.ds(base, bw)], idx_v)
        copies = []
        for c in range(bw // 128):
            copies.append(pltpu.async_copy(
                table_hbm.at[idx_v.at[pl.ds(c * 128, 128)]],
                rows_v.at[pl.ds(c * 128, 128)], sem))
        for cp in copies:
            cp.wait()
        pltpu.sync_copy(rows_v, out_hbm.at[pl.ds(base, bw)])

    return gather_k(emb, idx)


def _loss_st_body(zp_ref, zq_ref, o_ref, loss_ref, acc_ref):
    i = pl.program_id(0)
    nb = pl.num_programs(0)
    zp = zp_ref[...].reshape(E_DIM, 1024)
    zq = jnp.transpose(zq_ref[...])
    diff = zq - zp
    o_ref[...] = (zp + diff).reshape(1, E_DIM, 1024)
    s = jnp.sum(diff * diff)

    @pl.when(i == 0)
    def _():
        acc_ref[0] = s

    @pl.when(i > 0)
    def _():
        acc_ref[0] += s

    @pl.when(i == nb - 1)
    def _():
        m = acc_ref[0] / jnp.float32(M * E_DIM)
        loss_ref[0, 0] = m + jnp.float32(BETA) * m


def _loss_st(z4, zq_flat):
    bm = 1024
    return pl.pallas_call(
        _loss_st_body,
        grid=(M // bm,),
        in_specs=[
            pl.BlockSpec((1, E_DIM, bm), lambda i: (i, 0, 0)),
            pl.BlockSpec((bm, E_DIM), lambda i: (i, 0)),
        ],
        out_specs=[
            pl.BlockSpec((1, E_DIM, bm), lambda i: (i, 0, 0)),
            pl.BlockSpec((1, 1), lambda i: (0, 0), memory_space=pltpu.SMEM),
        ],
        out_shape=[
            jax.ShapeDtypeStruct((8, E_DIM, bm), jnp.float32),
            jax.ShapeDtypeStruct((1, 1), jnp.float32),
        ],
        scratch_shapes=[pltpu.SMEM((1,), jnp.float32)],
    )(z4, zq_flat)


def kernel(z, embedding_weight):
    z4 = z.reshape(8, E_DIM, 1024)
    z_flat = jnp.transpose(z, (0, 2, 3, 1)).reshape(-1, E_DIM)
    esum = _esum(embedding_weight)
    d, idx2 = _dist_argmin(z_flat, embedding_weight.astype(jnp.bfloat16),
                           esum)
    idx = idx2.reshape(M)
    zq_flat = _sc_gather(embedding_weight, idx)
    out4, loss2 = _loss_st(z4, zq_flat)
    z_q = out4.reshape(z.shape)
    return (z_q, loss2[0, 0], idx, d)


# speedup vs baseline: 1.1495x; 1.0959x over previous
import functools

import jax
import jax.numpy as jnp
from jax import lax
from jax.experimental import pallas as pl
from jax.experimental.pallas import tpu as pltpu
from jax.experimental.pallas import tpu_sc as plsc

N_E = 8192
E_DIM = 256
BETA = 0.25

M = 8192
BM = 256
BN = 2048
NJ = N_E // BN


RB = 64


def _esum_body(e_ref, out_ref):
    eb = e_ref[...]
    ones = jnp.ones((1, E_DIM), jnp.float32)
    out_ref[...] = lax.dot_general(ones, eb * eb, (((1,), (1,)), ((), ())),
                                   preferred_element_type=jnp.float32)


def _esum(emb):
    bn = 1024
    return pl.pallas_call(
        _esum_body,
        grid=(N_E // bn,),
        in_specs=[pl.BlockSpec((bn, E_DIM), lambda i: (i, 0))],
        out_specs=pl.BlockSpec((1, bn), lambda i: (0, i)),
        out_shape=jax.ShapeDtypeStruct((1, N_E), jnp.float32),
    )(emb)


def _dist_argmin_body(z_ref, e_ref, es_ref, d_ref, idx_ref):
    zb = z_ref[...]
    eb = e_ref[...]

    zb2 = (zb + zb).astype(jnp.bfloat16)
    mm2 = lax.dot_general(zb2, eb, (((1,), (1,)), ((), ())),
                          preferred_element_type=jnp.float32)
    zsum = jnp.sum(zb * zb, axis=1, keepdims=True)
    esum = es_ref[...]

    nc = N_E // 128
    for r in range(BM // RB):
        rl, rh = r * RB, (r + 1) * RB
        zs = zsum[rl:rh]
        acc_v = jnp.full((RB, 128), jnp.inf, jnp.float32)
        acc_i = jnp.zeros((RB, 128), jnp.int32)
        for c in range(nc):
            lo, hi = c * 128, (c + 1) * 128
            v = (zs + esum[:, lo:hi]) - mm2[rl:rh, lo:hi]
            d_ref[rl:rh, lo:hi] = v
            m = v < acc_v
            acc_i = jnp.where(m, jnp.full((RB, 128), c, jnp.int32), acc_i)
            acc_v = jnp.where(m, v, acc_v)
        rowmin = jnp.min(acc_v, axis=1, keepdims=True)
        gidx = acc_i * 128 + lax.broadcasted_iota(jnp.int32, (RB, 128), 1)
        idx_ref[rl:rh] = jnp.min(
            jnp.where(acc_v == rowmin, gidx, jnp.int32(2**30)),
            axis=1, keepdims=True)


def _dist_argmin(z_flat, emb_bf, esum):
    return pl.pallas_call(
        _dist_argmin_body,
        grid=(M // BM,),
        in_specs=[
            pl.BlockSpec((BM, E_DIM), lambda i: (i, 0)),
            pl.BlockSpec((N_E, E_DIM), lambda i: (0, 0)),
            pl.BlockSpec((1, N_E), lambda i: (0, 0)),
        ],
        out_specs=[
            pl.BlockSpec((BM, N_E), lambda i: (i, 0)),
            pl.BlockSpec((BM, 1), lambda i: (i, 0)),
        ],
        out_shape=[
            jax.ShapeDtypeStruct((M, N_E), jnp.float32),
            jax.ShapeDtypeStruct((M, 1), jnp.int32),
        ],
    )(z_flat, emb_bf, esum)


def _sc_gather(emb, idx):
    info = plsc.get_sparse_core_info()
    nw = info.num_cores * info.num_subcores
    bw = M // nw
    mesh = plsc.VectorSubcoreMesh(core_axis_name="c", subcore_axis_name="s")

    @functools.partial(
        pl.kernel, mesh=mesh,
        out_type=jax.ShapeDtypeStruct((M, E_DIM), jnp.float32),
        scratch_types=[
            pltpu.VMEM((bw,), jnp.int32),
            pltpu.VMEM((bw, E_DIM), jnp.float32),
            pltpu.SemaphoreType.DMA,
        ],
    )
    def gather_k(table_hbm, idx_hbm, out_hbm, idx_v, rows_v, sem):
        wid = lax.axis_index("s") * info.num_cores + lax.axis_index("c")
        base = wid * bw
        pltpu.sync_copy(idx_hbm.at[pl.ds(base, bw)], idx_v)
        copies = []
        for c in range(bw // 128):
            copies.append(pltpu.async_copy(
                table_hbm.at[idx_v.at[pl.ds(c * 128, 128)]],
                rows_v.at[pl.ds(c * 128, 128)], sem))
        for cp in copies:
            cp.wait()
        pltpu.sync_copy(rows_v, out_hbm.at[pl.ds(base, bw)])

    return gather_k(emb, idx)


def _loss_st_body(zp_ref, zq_ref, o_ref, loss_ref, acc_ref):
    i = pl.program_id(0)
    nb = pl.num_programs(0)
    zp = zp_ref[...]
    zq = zq_ref[...]
    diff = zq - zp
    o_ref[...] = zp + diff
    s = jnp.sum(diff * diff)

    @pl.when(i == 0)
    def _():
        acc_ref[0] = s

    @pl.when(i > 0)
    def _():
        acc_ref[0] += s

    @pl.when(i == nb - 1)
    def _():
        m = acc_ref[0] / jnp.float32(M * E_DIM)
        loss_ref[0, 0] = m + jnp.float32(BETA) * m


def _loss_st(zp_flat, zq_flat):
    bm = 1024
    return pl.pallas_call(
        _loss_st_body,
        grid=(M // bm,),
        in_specs=[
            pl.BlockSpec((bm, E_DIM), lambda i: (i, 0)),
            pl.BlockSpec((bm, E_DIM), lambda i: (i, 0)),
        ],
        out_specs=[
            pl.BlockSpec((bm, E_DIM), lambda i: (i, 0)),
            pl.BlockSpec((1, 1), lambda i: (0, 0), memory_space=pltpu.SMEM),
        ],
        out_shape=[
            jax.ShapeDtypeStruct((M, E_DIM), jnp.float32),
            jax.ShapeDtypeStruct((1, 1), jnp.float32),
        ],
        scratch_shapes=[pltpu.SMEM((1,), jnp.float32)],
    )(zp_flat, zq_flat)


def kernel(z, embedding_weight):
    zp = jnp.transpose(z, (0, 2, 3, 1))
    z_flat = zp.reshape(-1, E_DIM)
    esum = _esum(embedding_weight)
    d, idx2 = _dist_argmin(z_flat, embedding_weight.astype(jnp.bfloat16),
                           esum)
    idx = idx2.reshape(M)
    zq_flat = _sc_gather(embedding_weight, idx)
    out_flat, loss2 = _loss_st(z_flat, zq_flat)
    z_q = jnp.transpose(out_flat.reshape(zp.shape), (0, 3, 1, 2))
    return (z_q, loss2[0, 0], idx, d)
